# baseline (speedup 1.0000x reference)
import jax
import jax.numpy as jnp
from jax.experimental import pallas as pl
from jax.experimental.pallas import tpu as pltpu


def _pool_body(x_ref, o_ref):
    rows_per, e, bb = x_ref.shape
    w_out = o_ref.shape[1]
    cols_per = e // w_out
    s = x_ref[...].sum(axis=0)
    t = s.reshape(w_out, cols_per, bb).sum(axis=1)
    o_ref[...] = (t * (1.0 / float(rows_per * cols_per))).reshape(1, w_out, bb)


@jax.jit
def _adaptive_pool(x):
    B, N, E = x.shape
    H, W = 4, 8
    rows_per = N // H

    xt = jnp.transpose(x, (1, 2, 0))

    bb = B
    for cand in (1024, 512, 256, 128):
        if B % cand == 0:
            bb = cand
            break

    cost = pl.CostEstimate(
        flops=B * N * E,
        transcendentals=0,
        bytes_accessed=B * N * E * 4 + B * H * W * 4,
    )
    out_t = pl.pallas_call(
        _pool_body,
        out_shape=jax.ShapeDtypeStruct((H, W, B), jnp.float32),
        grid=(H, B // bb),
        in_specs=[pl.BlockSpec((rows_per, E, bb), lambda i, j: (i, 0, j))],
        out_specs=pl.BlockSpec((1, W, bb), lambda i, j: (i, 0, j)),
        compiler_params=pltpu.CompilerParams(
            dimension_semantics=("arbitrary", "arbitrary"),
        ),
        cost_estimate=cost,
    )(xt)
    return jnp.transpose(out_t.reshape(H * W, B)).astype(x.dtype)


def kernel(x):
    return _adaptive_pool(x)


# speedup vs baseline: 1.1449x; 1.1449x over previous
import jax
import jax.numpy as jnp
from jax.experimental import pallas as pl
from jax.experimental.pallas import tpu as pltpu


def _pool_body(x_ref, o_ref):
    n, e, bb = x_ref.shape
    h_out, w_out = o_ref.shape[0], o_ref.shape[1]
    rows_per = n // h_out
    cols_per = e // w_out
    v = x_ref[...]
    s = v.reshape(h_out, rows_per, e, bb).sum(axis=1)
    t = s.reshape(h_out, w_out, cols_per, bb).sum(axis=2)
    o_ref[...] = t * (1.0 / float(rows_per * cols_per))


@jax.jit
def _adaptive_pool(x):
    B, N, E = x.shape
    H, W = 4, 8

    xt = jnp.transpose(x, (1, 2, 0))

    bb = B
    for cand in (512, 256, 128):
        if B % cand == 0:
            bb = cand
            break
    n_blocks = B // bb

    cost = pl.CostEstimate(
        flops=B * N * E,
        transcendentals=0,
        bytes_accessed=B * N * E * 4 + B * H * W * 4,
    )
    out_t = pl.pallas_call(
        _pool_body,
        out_shape=jax.ShapeDtypeStruct((H, W, B), jnp.float32),
        grid=(n_blocks,),
        in_specs=[pl.BlockSpec((N, E, bb), lambda i: (0, 0, i))],
        out_specs=pl.BlockSpec((H, W, bb), lambda i: (0, 0, i)),
        compiler_params=pltpu.CompilerParams(
            dimension_semantics=("arbitrary",),
        ),
        cost_estimate=cost,
    )(xt)
    return jnp.transpose(out_t.reshape(H * W, B)).astype(x.dtype)


def kernel(x):
    return _adaptive_pool(x)
